# Initial kernel scaffold; baseline (speedup 1.0000x reference)
import jax
import jax.numpy as jnp
from jax.experimental import pallas as pl


def kernel(x, edge_index, W1, b1, W2, b2, W3, b3, Wr, br):
    raise NotImplementedError("write your pallas kernel here")


# speedup vs baseline: 10.2137x; 10.2137x over previous
import functools

import jax
import jax.numpy as jnp
from jax import lax
from jax.experimental import pallas as pl
from jax.experimental.pallas import tpu as pltpu
from jax.experimental.pallas import tpu_sc as plsc

NC = 2
NS = 16
CHUNK = 128

_MESH = dict(core_axis_name="c", subcore_axis_name="s", num_cores=NC,
             num_subcores=NS)


def _make_layer_kernel(n, e, dh, edge_split, const_rows=False):
    nchunks = e // CHUNK
    rpt = n // NS
    if edge_split:
        nw = NC * NS
    else:
        nw = NS
    per, rem = nchunks // nw, nchunks % nw

    @functools.partial(
        pl.kernel,
        out_type=jax.ShapeDtypeStruct((NC, n, dh), jnp.float32),
        mesh=plsc.VectorSubcoreMesh(**_MESH),
        scratch_types=[
            pltpu.VMEM((CHUNK,), jnp.int32),
            pltpu.VMEM((CHUNK,), jnp.int32),
            pltpu.VMEM((CHUNK,), jnp.int32),
            pltpu.VMEM((CHUNK, dh), jnp.float32),
            pltpu.VMEM_SHARED((n, dh), jnp.float32),
            pltpu.SemaphoreType.DMA,
        ],
    )
    def layer_kernel(g_hbm, src_hbm, dst_hbm, zeros_hbm, out_hbm, srcb,
                     srca, dstb, rows, acc, sem):
        c = lax.axis_index("c")
        s = lax.axis_index("s")
        if const_rows:
            w = c * NS + s
            pltpu.sync_copy(g_hbm, rows)
            pltpu.sync_copy(zeros_hbm.at[pl.ds(s * rpt, rpt)],
                            acc.at[pl.ds(s * rpt, rpt)])
        elif edge_split:
            w = c * NS + s
            @pl.when(c == 0)
            def _():
                pltpu.sync_copy(g_hbm.at[pl.ds(s * rpt, rpt)],
                                acc.at[pl.ds(s * rpt, rpt)])

            @pl.when(c != 0)
            def _():
                pltpu.sync_copy(zeros_hbm.at[pl.ds(s * rpt, rpt)],
                                acc.at[pl.ds(s * rpt, rpt)])
        else:
            w = s
            pltpu.sync_copy(g_hbm.at[pl.ds(c * n + s * rpt, rpt)],
                            acc.at[pl.ds(s * rpt, rpt)])
        plsc.subcore_barrier()
        nloc = per + (w < rem).astype(jnp.int32)

        def body(i, carry):
            off = pl.multiple_of((w + nw * i) * CHUNK, CHUNK)
            pltpu.sync_copy(dst_hbm.at[pl.ds(off, CHUNK)], dstb)
            if const_rows:
                pass
            elif edge_split:
                pltpu.sync_copy(src_hbm.at[pl.ds(off, CHUNK)], srcb)
                pltpu.async_copy(g_hbm.at[srcb], rows, sem).wait()
            else:
                pltpu.sync_copy(src_hbm.at[pl.ds(off, CHUNK)], srcb)
                base = c * n
                for j in range(CHUNK // 16):
                    srca[pl.ds(j * 16, 16)] = srcb[pl.ds(j * 16, 16)] + base
                pltpu.async_copy(g_hbm.at[srca], rows, sem).wait()
            pltpu.sync_copy(rows, acc.at[dstb], add=True)
            return carry

        lax.fori_loop(0, nloc, body, 0)
        plsc.subcore_barrier()
        pltpu.sync_copy(acc.at[pl.ds(s * rpt, rpt)],
                        out_hbm.at[c, pl.ds(s * rpt, rpt)])

    return layer_kernel


def _dis_from_hist(hist_blk):
    deg = 1.0 + hist_blk[0, :, 0:1] + hist_blk[1, :, 0:1]
    return lax.rsqrt(deg)


def _tc1_body(x_ref, w_ref, hist_ref, o_ref):
    dis = _dis_from_hist(hist_ref[...])
    o_ref[...] = dis * jnp.dot(x_ref[...], w_ref[...],
                               preferred_element_type=jnp.float32)


def _tc_mid_body(acc_ref, b_ref, w_ref, hist_ref, o_ref):
    dis = _dis_from_hist(hist_ref[...])
    a = jnp.concatenate([acc_ref[0], acc_ref[1]], axis=1)
    h = jax.nn.relu(dis * a + b_ref[...])
    o_ref[...] = dis * jnp.dot(h, w_ref[...],
                               preferred_element_type=jnp.float32)


def _tc_head_body(acc_ref, b_ref, wr_ref, br_ref, hist_ref, h_ref, r_ref):
    dis = _dis_from_hist(hist_ref[...])
    a = acc_ref[0] + acc_ref[1]
    h = jax.nn.relu(dis * a + b_ref[...])
    h_ref[...] = h
    r_ref[...] = jnp.dot(h, wr_ref[...],
                         preferred_element_type=jnp.float32) + br_ref[...]


def kernel(x, edge_index, W1, b1, W2, b2, W3, b3, Wr, br):
    n, d_in = x.shape
    e = edge_index.shape[1]
    dh = W1.shape[1]
    dhh = dh // 2
    dout = W3.shape[1]

    src = edge_index[0]
    dst = edge_index[1]
    npad = 10240
    zeros_nh = jnp.zeros((npad, dout), jnp.float32)
    ones_ch = jnp.ones((CHUNK, dout), jnp.float32)

    hist = _make_layer_kernel(npad, e, dout, True, const_rows=True)(
        ones_ch, src, dst, zeros_nh)

    bm = 1024
    nb = npad // bm
    hist_spec = pl.BlockSpec((NC, bm, dout), lambda h, i: (0, i, 0))

    g1 = pl.pallas_call(
        _tc1_body,
        grid=(2, nb),
        in_specs=[
            pl.BlockSpec((bm, d_in), lambda h, i: (i, 0)),
            pl.BlockSpec((d_in, dhh), lambda h, i: (0, h)),
            hist_spec,
        ],
        out_specs=pl.BlockSpec((bm, dhh), lambda h, i: (h * nb + i, 0)),
        out_shape=jax.ShapeDtypeStruct((2 * npad, dhh), jnp.float32),
    )(x, W1, hist)

    acc1 = _make_layer_kernel(npad, e, dhh, False)(g1, src, dst, zeros_nh)

    g2 = pl.pallas_call(
        _tc_mid_body,
        grid=(2, nb),
        in_specs=[
            pl.BlockSpec((NC, bm, dhh), lambda h, i: (0, i, 0)),
            pl.BlockSpec((1, dh), lambda h, i: (0, 0)),
            pl.BlockSpec((dh, dhh), lambda h, i: (0, h)),
            hist_spec,
        ],
        out_specs=pl.BlockSpec((bm, dhh), lambda h, i: (h * nb + i, 0)),
        out_shape=jax.ShapeDtypeStruct((2 * npad, dhh), jnp.float32),
    )(acc1, b1.reshape(1, dh), W2, hist)
    acc2 = _make_layer_kernel(npad, e, dhh, False)(g2, src, dst, zeros_nh)

    g3 = pl.pallas_call(
        _tc_mid_body,
        grid=(nb,),
        in_specs=[
            pl.BlockSpec((NC, bm, dhh), lambda i: (0, i, 0)),
            pl.BlockSpec((1, dh), lambda i: (0, 0)),
            pl.BlockSpec((dh, dout), lambda i: (0, 0)),
            pl.BlockSpec((NC, bm, dout), lambda i: (0, i, 0)),
        ],
        out_specs=pl.BlockSpec((bm, dout), lambda i: (i, 0)),
        out_shape=jax.ShapeDtypeStruct((npad, dout), jnp.float32),
    )(acc2, b2.reshape(1, dh), W3, hist)
    acc3 = _make_layer_kernel(npad, e, dout, True)(g3, src, dst, zeros_nh)

    h3, recon = pl.pallas_call(
        _tc_head_body,
        grid=(nb,),
        in_specs=[
            pl.BlockSpec((NC, bm, dout), lambda i: (0, i, 0)),
            pl.BlockSpec((1, dout), lambda i: (0, 0)),
            pl.BlockSpec((dout, d_in), lambda i: (0, 0)),
            pl.BlockSpec((1, d_in), lambda i: (0, 0)),
            pl.BlockSpec((NC, bm, dout), lambda i: (0, i, 0)),
        ],
        out_specs=[
            pl.BlockSpec((bm, dout), lambda i: (i, 0)),
            pl.BlockSpec((bm, d_in), lambda i: (i, 0)),
        ],
        out_shape=[
            jax.ShapeDtypeStruct((n, dout), jnp.float32),
            jax.ShapeDtypeStruct((n, d_in), jnp.float32),
        ],
    )(acc3, b3.reshape(1, dout), Wr, br.reshape(1, d_in), hist)

    return (h3, recon)
